# baseline (speedup 1.0000x reference)
import functools

import jax
import jax.numpy as jnp
from jax.experimental import pallas as pl
from jax.experimental.pallas import tpu as pltpu

BLK = 512


def _fused_body(x1t_ref, x2tn_ref, p1_ref, p2_ref, w1t_ref, b1_ref,
                w2t_ref, b2_ref, out_ref):
    n2 = x2tn_ref.shape[2]
    x1t = x1t_ref[0]
    x2tn = x2tn_ref[0]

    cross = jax.lax.dot_general(
        x1t, x2tn, (((0,), (0,)), ((), ())),
        precision=jax.lax.Precision.HIGHEST,
        preferred_element_type=jnp.float32)
    sq1 = jnp.sum(x1t * x1t, axis=0).reshape(BLK, 1)
    sq2 = jnp.sum(x2tn * x2tn, axis=0).reshape(1, n2) * 0.25
    d2 = (cross + sq1) + sq2

    col = jax.lax.broadcasted_iota(jnp.int32, (BLK, n2), 1)
    big = jnp.float32(jnp.inf)

    def rowmin(x):
        return jnp.min(x, axis=1).reshape(BLK, 1)

    m1 = rowmin(d2)
    a1 = rowmin(jnp.where(d2 == m1, col, n2))
    t2 = jnp.where(col == a1, big, d2)
    m2 = rowmin(t2)
    a2 = rowmin(jnp.where(t2 == m2, col, n2))
    t3 = jnp.where(col == a2, big, t2)
    m3 = rowmin(t3)
    a3 = rowmin(jnp.where(t3 == m3, col, n2))

    r1 = 1.0 / jnp.maximum(m1, jnp.float32(1e-8))
    r2 = 1.0 / jnp.maximum(m2, jnp.float32(1e-8))
    r3 = 1.0 / jnp.maximum(m3, jnp.float32(1e-8))
    inv_norm = 1.0 / (r1 + r2 + r3)
    zero = jnp.float32(0.0)
    s = jnp.where(col == a1, r1 * inv_norm,
                  jnp.where(col == a2, r2 * inv_norm,
                            jnp.where(col == a3, r3 * inv_norm, zero)))

    interp = jnp.dot(s, p2_ref[0], preferred_element_type=jnp.float32)

    w1t = w1t_ref[...]
    c2 = interp.shape[1]
    h = jnp.dot(interp, w1t[:c2, :], preferred_element_type=jnp.float32)
    h = h + jnp.dot(p1_ref[0], w1t[c2:, :], preferred_element_type=jnp.float32)
    h = jnp.maximum(h + b1_ref[...], 0.0)
    o = jnp.dot(h, w2t_ref[...], preferred_element_type=jnp.float32)
    o = jnp.maximum(o + b2_ref[...], 0.0)
    out_ref[0] = o


@jax.jit
def kernel(xyz1, xyz2, points1, points2, W1, b1, W2, b2):
    B, n1, _ = xyz1.shape
    n2 = xyz2.shape[1]
    c1 = points1.shape[2]
    c2 = points2.shape[2]
    cout = W2.shape[0]

    x1t = jnp.swapaxes(xyz1, 1, 2)
    x2tn = jnp.swapaxes(xyz2, 1, 2) * jnp.float32(-2.0)
    w1t = W1.T
    w2t = W2.T
    b1r = b1.reshape(1, -1)
    b2r = b2.reshape(1, -1)

    grid = (B, n1 // BLK)
    out = pl.pallas_call(
        _fused_body,
        grid=grid,
        in_specs=[
            pl.BlockSpec((1, 3, BLK), lambda b, i: (b, 0, i)),
            pl.BlockSpec((1, 3, n2), lambda b, i: (b, 0, 0)),
            pl.BlockSpec((1, BLK, c1), lambda b, i: (b, i, 0)),
            pl.BlockSpec((1, n2, c2), lambda b, i: (b, 0, 0)),
            pl.BlockSpec((c2 + c1, cout), lambda b, i: (0, 0)),
            pl.BlockSpec((1, cout), lambda b, i: (0, 0)),
            pl.BlockSpec((cout, cout), lambda b, i: (0, 0)),
            pl.BlockSpec((1, cout), lambda b, i: (0, 0)),
        ],
        out_specs=pl.BlockSpec((1, BLK, cout), lambda b, i: (b, i, 0)),
        out_shape=jax.ShapeDtypeStruct((B, n1, cout), jnp.float32),
        compiler_params=pltpu.CompilerParams(
            dimension_semantics=("parallel", "parallel"),
        ),
    )(x1t, x2tn, points1, points2, w1t, b1r, w2t, b2r)
    return out


# speedup vs baseline: 1.2895x; 1.2895x over previous
import functools

import jax
import jax.numpy as jnp
from jax.experimental import pallas as pl
from jax.experimental.pallas import tpu as pltpu

BLK = 512


def _fused_body(x1t_ref, x2tn_ref, p1_ref, p2_ref, w1t_ref, b1_ref,
                w2t_ref, b2_ref, out_ref):
    n2 = x2tn_ref.shape[2]
    x1t = x1t_ref[0]
    x2tn = x2tn_ref[0]

    d0 = x1t[0, :].reshape(BLK, 1) + x2tn[0, :].reshape(1, n2)
    d1 = x1t[1, :].reshape(BLK, 1) + x2tn[1, :].reshape(1, n2)
    dd = x1t[2, :].reshape(BLK, 1) + x2tn[2, :].reshape(1, n2)
    d2 = d0 * d0 + d1 * d1 + dd * dd

    col = jax.lax.broadcasted_iota(jnp.int32, (BLK, n2), 1)
    big = jnp.float32(jnp.inf)

    def rowmin(x):
        return jnp.min(x, axis=1).reshape(BLK, 1)

    m1 = rowmin(d2)
    a1 = rowmin(jnp.where(d2 == m1, col, n2))
    t2 = jnp.where(col == a1, big, d2)
    m2 = rowmin(t2)
    a2 = rowmin(jnp.where(t2 == m2, col, n2))
    t3 = jnp.where(col == a2, big, t2)
    m3 = rowmin(t3)
    a3 = rowmin(jnp.where(t3 == m3, col, n2))

    r1 = 1.0 / jnp.maximum(m1, jnp.float32(1e-8))
    r2 = 1.0 / jnp.maximum(m2, jnp.float32(1e-8))
    r3 = 1.0 / jnp.maximum(m3, jnp.float32(1e-8))
    inv_norm = 1.0 / (r1 + r2 + r3)
    zero = jnp.float32(0.0)
    s = jnp.where(col == a1, r1 * inv_norm,
                  jnp.where(col == a2, r2 * inv_norm,
                            jnp.where(col == a3, r3 * inv_norm, zero)))

    interp = jnp.dot(s, p2_ref[0], preferred_element_type=jnp.float32)

    w1t = w1t_ref[...]
    c2 = interp.shape[1]
    h = jnp.dot(interp, w1t[:c2, :], preferred_element_type=jnp.float32)
    h = h + jnp.dot(p1_ref[0], w1t[c2:, :], preferred_element_type=jnp.float32)
    h = jnp.maximum(h + b1_ref[...], 0.0)
    o = jnp.dot(h, w2t_ref[...], preferred_element_type=jnp.float32)
    o = jnp.maximum(o + b2_ref[...], 0.0)
    out_ref[0] = o


@jax.jit
def kernel(xyz1, xyz2, points1, points2, W1, b1, W2, b2):
    B, n1, _ = xyz1.shape
    n2 = xyz2.shape[1]
    c1 = points1.shape[2]
    c2 = points2.shape[2]
    cout = W2.shape[0]

    x1t = jnp.swapaxes(xyz1, 1, 2)
    x2tn = -jnp.swapaxes(xyz2, 1, 2)
    w1t = W1.T
    w2t = W2.T
    b1r = b1.reshape(1, -1)
    b2r = b2.reshape(1, -1)

    grid = (B, n1 // BLK)
    out = pl.pallas_call(
        _fused_body,
        grid=grid,
        in_specs=[
            pl.BlockSpec((1, 3, BLK), lambda b, i: (b, 0, i)),
            pl.BlockSpec((1, 3, n2), lambda b, i: (b, 0, 0)),
            pl.BlockSpec((1, BLK, c1), lambda b, i: (b, i, 0)),
            pl.BlockSpec((1, n2, c2), lambda b, i: (b, 0, 0)),
            pl.BlockSpec((c2 + c1, cout), lambda b, i: (0, 0)),
            pl.BlockSpec((1, cout), lambda b, i: (0, 0)),
            pl.BlockSpec((cout, cout), lambda b, i: (0, 0)),
            pl.BlockSpec((1, cout), lambda b, i: (0, 0)),
        ],
        out_specs=pl.BlockSpec((1, BLK, cout), lambda b, i: (b, i, 0)),
        out_shape=jax.ShapeDtypeStruct((B, n1, cout), jnp.float32),
        compiler_params=pltpu.CompilerParams(
            dimension_semantics=("parallel", "parallel"),
        ),
    )(x1t, x2tn, points1, points2, w1t, b1r, w2t, b2r)
    return out


# speedup vs baseline: 1.5136x; 1.1737x over previous
import functools

import jax
import jax.numpy as jnp
from jax.experimental import pallas as pl
from jax.experimental.pallas import tpu as pltpu

BLK = 512


def _fused_body(x1t_ref, x2tn_ref, p1_ref, p2_ref, w1t_ref, b1_ref,
                w2t_ref, b2_ref, out_ref):
    n2 = x2tn_ref.shape[2]
    x1t = x1t_ref[0]
    x2tn = x2tn_ref[0]

    d0 = x1t[0, :].reshape(BLK, 1) + x2tn[0, :].reshape(1, n2)
    d1 = x1t[1, :].reshape(BLK, 1) + x2tn[1, :].reshape(1, n2)
    dd = x1t[2, :].reshape(BLK, 1) + x2tn[2, :].reshape(1, n2)
    d2 = d0 * d0 + d1 * d1 + dd * dd

    colf = jax.lax.broadcasted_iota(
        jnp.int32, (BLK, n2), 1).astype(jnp.float32)
    big = jnp.float32(jnp.inf)
    nf = jnp.float32(n2)

    def rowmin(x):
        return jnp.min(x, axis=1).reshape(BLK, 1)

    m1 = rowmin(d2)
    a1 = rowmin(jnp.where(d2 == m1, colf, nf))
    c1 = colf == a1
    t2 = jnp.where(c1, big, d2)
    m2 = rowmin(t2)
    a2 = rowmin(jnp.where(t2 == m2, colf, nf))
    c2 = colf == a2
    t3 = jnp.where(c2, big, t2)
    m3 = rowmin(t3)
    a3 = rowmin(jnp.where(t3 == m3, colf, nf))
    c3 = colf == a3

    r1 = 1.0 / jnp.maximum(m1, jnp.float32(1e-8))
    r2 = 1.0 / jnp.maximum(m2, jnp.float32(1e-8))
    r3 = 1.0 / jnp.maximum(m3, jnp.float32(1e-8))
    inv_norm = 1.0 / (r1 + r2 + r3)
    zero = jnp.float32(0.0)
    s = jnp.where(c1, r1 * inv_norm,
                  jnp.where(c2, r2 * inv_norm,
                            jnp.where(c3, r3 * inv_norm, zero)))

    interp = jnp.dot(s, p2_ref[0], preferred_element_type=jnp.float32)

    w1t = w1t_ref[...]
    c2 = interp.shape[1]
    h = jnp.dot(interp, w1t[:c2, :], preferred_element_type=jnp.float32)
    h = h + jnp.dot(p1_ref[0], w1t[c2:, :], preferred_element_type=jnp.float32)
    h = jnp.maximum(h + b1_ref[...], 0.0)
    o = jnp.dot(h, w2t_ref[...], preferred_element_type=jnp.float32)
    o = jnp.maximum(o + b2_ref[...], 0.0)
    out_ref[0] = o


@jax.jit
def kernel(xyz1, xyz2, points1, points2, W1, b1, W2, b2):
    B, n1, _ = xyz1.shape
    n2 = xyz2.shape[1]
    c1 = points1.shape[2]
    c2 = points2.shape[2]
    cout = W2.shape[0]

    x1t = jnp.swapaxes(xyz1, 1, 2)
    x2tn = -jnp.swapaxes(xyz2, 1, 2)
    w1t = W1.T
    w2t = W2.T
    b1r = b1.reshape(1, -1)
    b2r = b2.reshape(1, -1)

    grid = (B, n1 // BLK)
    out = pl.pallas_call(
        _fused_body,
        grid=grid,
        in_specs=[
            pl.BlockSpec((1, 3, BLK), lambda b, i: (b, 0, i)),
            pl.BlockSpec((1, 3, n2), lambda b, i: (b, 0, 0)),
            pl.BlockSpec((1, BLK, c1), lambda b, i: (b, i, 0)),
            pl.BlockSpec((1, n2, c2), lambda b, i: (b, 0, 0)),
            pl.BlockSpec((c2 + c1, cout), lambda b, i: (0, 0)),
            pl.BlockSpec((1, cout), lambda b, i: (0, 0)),
            pl.BlockSpec((cout, cout), lambda b, i: (0, 0)),
            pl.BlockSpec((1, cout), lambda b, i: (0, 0)),
        ],
        out_specs=pl.BlockSpec((1, BLK, cout), lambda b, i: (b, i, 0)),
        out_shape=jax.ShapeDtypeStruct((B, n1, cout), jnp.float32),
        compiler_params=pltpu.CompilerParams(
            dimension_semantics=("parallel", "parallel"),
        ),
    )(x1t, x2tn, points1, points2, w1t, b1r, w2t, b2r)
    return out
